# baseline (speedup 1.0000x reference)
import dataclasses
import functools

import jax
import jax.numpy as jnp
from jax import lax
from jax.experimental import pallas as pl
from jax.experimental.pallas import tpu as pltpu
from jax.experimental.pallas import tpu_sc as plsc

_DIM = 2048
_E = 16
_T = 16384
_LANES = 16
_NC = 2
_NS = 16
_NW = _NC * _NS
_TPW = _T // _NW
_BT = 512


def _tc_logits_body(w_ref, x_ref, out_ref):
    out_ref[...] = lax.dot_general(
        w_ref[...], x_ref[...], (((1,), (1,)), ((), ())),
        preferred_element_type=jnp.float32)


def _tc_logits(x, w, start_block, nblocks):
    return pl.pallas_call(
        _tc_logits_body,
        grid=(nblocks,),
        in_specs=[
            pl.BlockSpec((_E, _DIM), lambda i: (0, 0)),
            pl.BlockSpec((_BT, _DIM), lambda i: (i + start_block, 0)),
        ],
        out_specs=pl.BlockSpec((_E, _BT), lambda i: (0, i)),
        out_shape=jax.ShapeDtypeStruct((_E, nblocks * _BT), jnp.float32),
    )(w, x)


def _router_body(lt_hbm, ow_hbm, oi_hbm, lt_v, w_v, i_v, *, tpw):
    wid = lax.axis_index("s") * _NC + lax.axis_index("c")
    base = wid * tpw
    pltpu.sync_copy(lt_hbm.at[:, pl.ds(base, tpw)], lt_v)

    neg = jnp.full((_LANES,), -3.0e38, jnp.float32)
    big = jnp.full((_LANES,), _E, jnp.int32)

    @pl.loop(0, tpw, step=_LANES)
    def _(j):
        ls = [lt_v[e, pl.ds(j, _LANES)] for e in range(_E)]
        m1 = ls[0]
        for e in range(1, _E):
            m1 = jnp.maximum(m1, ls[e])
        i1 = big
        for e in range(_E):
            i1 = jnp.minimum(i1, jnp.where(ls[e] == m1, jnp.int32(e), jnp.int32(_E)))
        m2 = neg
        for e in range(_E):
            m2 = jnp.maximum(m2, jnp.where(i1 == e, neg, ls[e]))
        i2 = big
        for e in range(_E):
            cond = (ls[e] == m2) & (i1 != e)
            i2 = jnp.minimum(i2, jnp.where(cond, jnp.int32(e), jnp.int32(_E)))
        t = jnp.exp(m2 - m1)
        w1 = 1.0 / (1.0 + t)
        w2 = t * w1
        w_v[0, pl.ds(j, _LANES)] = w1
        w_v[1, pl.ds(j, _LANES)] = w2
        i_v[0, pl.ds(j, _LANES)] = i1
        i_v[1, pl.ds(j, _LANES)] = i2

    pltpu.sync_copy(w_v, ow_hbm.at[:, pl.ds(base, tpw)])
    pltpu.sync_copy(i_v, oi_hbm.at[:, pl.ds(base, tpw)])


def _sc_router(lt):
    tc = lt.shape[1]
    tpw = tc // _NW
    mesh = plsc.VectorSubcoreMesh(core_axis_name="c", subcore_axis_name="s")
    cp = pltpu.CompilerParams()
    if "needs_layout_passes" in pltpu.CompilerParams.__dataclass_fields__:
        cp = dataclasses.replace(cp, needs_layout_passes=False)
    f = pl.kernel(
        functools.partial(_router_body, tpw=tpw),
        out_type=(
            jax.ShapeDtypeStruct((2, tc), jnp.float32),
            jax.ShapeDtypeStruct((2, tc), jnp.int32),
        ),
        mesh=mesh,
        scratch_types=[
            pltpu.VMEM((_E, tpw), jnp.float32),
            pltpu.VMEM((2, tpw), jnp.float32),
            pltpu.VMEM((2, tpw), jnp.int32),
        ],
        compiler_params=cp,
    )
    return f(lt)


_CHUNK_BLOCKS = (_T // _BT,)


@jax.jit
def kernel(x, W):
    wpls, ipls = [], []
    start = 0
    for nb in _CHUNK_BLOCKS:
        lt = _tc_logits(x, W, start, nb)
        wpl, ipl = _sc_router(lt)
        wpls.append(wpl)
        ipls.append(ipl)
        start += nb
    wcat = wpls[0] if len(wpls) == 1 else jnp.concatenate(wpls, axis=1)
    icat = ipls[0] if len(ipls) == 1 else jnp.concatenate(ipls, axis=1)
    return wcat.T, icat.T


# speedup vs baseline: 1.1326x; 1.1326x over previous
import dataclasses
import functools

import jax
import jax.numpy as jnp
from jax import lax
from jax.experimental import pallas as pl
from jax.experimental.pallas import tpu as pltpu
from jax.experimental.pallas import tpu_sc as plsc

_DIM = 2048
_E = 16
_T = 16384
_LANES = 16
_NC = 2
_NS = 16
_NW = _NC * _NS
_TPW = _T // _NW
_BT = 1024


def _tc_logits_body(w_ref, x_ref, out_ref):
    out_ref[...] = lax.dot_general(
        w_ref[...], x_ref[...], (((1,), (1,)), ((), ())),
        preferred_element_type=jnp.float32)


def _tc_logits(x, w, start_block, nblocks):
    return pl.pallas_call(
        _tc_logits_body,
        grid=(nblocks,),
        in_specs=[
            pl.BlockSpec((_E, _DIM), lambda i: (0, 0)),
            pl.BlockSpec((_BT, _DIM), lambda i: (i + start_block, 0)),
        ],
        out_specs=pl.BlockSpec((_E, _BT), lambda i: (0, i)),
        out_shape=jax.ShapeDtypeStruct((_E, nblocks * _BT), jnp.float32),
    )(w, x)


def _router_body(lt_hbm, ow_hbm, oi_hbm, lt_v, w_v, i_v, *, tpw):
    wid = lax.axis_index("s") * _NC + lax.axis_index("c")
    base = wid * tpw
    pltpu.sync_copy(lt_hbm.at[:, pl.ds(base, tpw)], lt_v)

    neg = jnp.full((_LANES,), -3.0e38, jnp.float32)
    big = jnp.full((_LANES,), _E, jnp.int32)

    @pl.loop(0, tpw, step=_LANES)
    def _(j):
        ls = [lt_v[e, pl.ds(j, _LANES)] for e in range(_E)]
        m1 = ls[0]
        for e in range(1, _E):
            m1 = jnp.maximum(m1, ls[e])
        i1 = big
        for e in range(_E):
            i1 = jnp.minimum(i1, jnp.where(ls[e] == m1, jnp.int32(e), jnp.int32(_E)))
        m2 = neg
        for e in range(_E):
            m2 = jnp.maximum(m2, jnp.where(i1 == e, neg, ls[e]))
        i2 = big
        for e in range(_E):
            cond = (ls[e] == m2) & (i1 != e)
            i2 = jnp.minimum(i2, jnp.where(cond, jnp.int32(e), jnp.int32(_E)))
        t = jnp.exp(m2 - m1)
        w1 = 1.0 / (1.0 + t)
        w2 = t * w1
        w_v[0, pl.ds(j, _LANES)] = w1
        w_v[1, pl.ds(j, _LANES)] = w2
        i_v[0, pl.ds(j, _LANES)] = i1
        i_v[1, pl.ds(j, _LANES)] = i2

    pltpu.sync_copy(w_v, ow_hbm.at[:, pl.ds(base, tpw)])
    pltpu.sync_copy(i_v, oi_hbm.at[:, pl.ds(base, tpw)])


def _sc_router(lt):
    tc = lt.shape[1]
    tpw = tc // _NW
    mesh = plsc.VectorSubcoreMesh(core_axis_name="c", subcore_axis_name="s")
    cp = pltpu.CompilerParams()
    if "needs_layout_passes" in pltpu.CompilerParams.__dataclass_fields__:
        cp = dataclasses.replace(cp, needs_layout_passes=False)
    f = pl.kernel(
        functools.partial(_router_body, tpw=tpw),
        out_type=(
            jax.ShapeDtypeStruct((2, tc), jnp.float32),
            jax.ShapeDtypeStruct((2, tc), jnp.int32),
        ),
        mesh=mesh,
        scratch_types=[
            pltpu.VMEM((_E, tpw), jnp.float32),
            pltpu.VMEM((2, tpw), jnp.float32),
            pltpu.VMEM((2, tpw), jnp.int32),
        ],
        compiler_params=cp,
    )
    return f(lt)


_CHUNK_BLOCKS = (_T // _BT,)


@jax.jit
def kernel(x, W):
    wpls, ipls = [], []
    start = 0
    for nb in _CHUNK_BLOCKS:
        lt = _tc_logits(x, W, start, nb)
        wpl, ipl = _sc_router(lt)
        wpls.append(wpl)
        ipls.append(ipl)
        start += nb
    wcat = wpls[0] if len(wpls) == 1 else jnp.concatenate(wpls, axis=1)
    icat = ipls[0] if len(ipls) == 1 else jnp.concatenate(ipls, axis=1)
    return wcat.T, icat.T
